# baseline (speedup 1.0000x reference)
import jax
import jax.numpy as jnp
from jax.experimental import pallas as pl


def _body(x_ref, td_ref, tw_ref, out_ref):
    T = td_ref.shape[0]
    N = x_ref.shape[1]
    col = x_ref[0]
    d_f = jnp.floor(jnp.clip(col[:, 1:2] * T, 0, T - 1))
    w_f = jnp.floor(jnp.clip(col[:, 2:3], 0, 6))
    c_col = d_f * 8.0 + w_f
    c_row = jnp.transpose(c_col, (1, 0)).astype(jnp.int32)
    d_idx = c_row >> 3
    w_idx = c_row & 7

    iota_t = jax.lax.broadcasted_iota(jnp.int32, (T, N), 0)
    oh_d = (iota_t == d_idx).astype(jnp.float32)
    iota_w = jax.lax.broadcasted_iota(jnp.int32, (8, N), 0)
    oh_w = (iota_w == w_idx).astype(jnp.float32)

    acc = jax.lax.dot_general(td_ref[...], oh_d, (((0,), (0,)), ((), ())),
                              preferred_element_type=jnp.float32)
    acc = acc + jax.lax.dot_general(tw_ref[...], oh_w, (((0,), (0,)), ((), ())),
                                    preferred_element_type=jnp.float32)
    out_ref[0, :, :] = acc


def kernel(x, time_day, time_week):
    B, S, N, _ = x.shape
    T, F = time_day.shape
    xl = x[:, -1]
    tw_pad = jnp.zeros((8, F), jnp.float32).at[:7].set(time_week)

    out = pl.pallas_call(
        _body,
        grid=(B,),
        in_specs=[
            pl.BlockSpec((1, N, 3), lambda b: (b, 0, 0)),
            pl.BlockSpec((T, F), lambda b: (0, 0)),
            pl.BlockSpec((8, F), lambda b: (0, 0)),
        ],
        out_specs=pl.BlockSpec((1, F, N), lambda b: (b, 0, 0)),
        out_shape=jax.ShapeDtypeStruct((B, F, N), jnp.float32),
    )(xl, time_day, tw_pad)
    return out[..., None]


# speedup vs baseline: 1.1006x; 1.1006x over previous
import jax
import jax.numpy as jnp
from jax.experimental import pallas as pl


def _body(x_ref, td_ref, tw_ref, out_ref):
    T = td_ref.shape[0]
    N = x_ref.shape[1]
    colT = jnp.transpose(x_ref[0], (1, 0))
    d_idx = jnp.clip(colT[1:2, :] * T, 0, T - 1).astype(jnp.int32)
    w_idx = jnp.clip(colT[2:3, :], 0, 6).astype(jnp.int32)

    iota_t = jax.lax.broadcasted_iota(jnp.int32, (T, N), 0)
    oh_d = (iota_t == d_idx).astype(jnp.float32)
    iota_w = jax.lax.broadcasted_iota(jnp.int32, (8, N), 0)
    oh_w = (iota_w == w_idx).astype(jnp.float32)

    acc = jax.lax.dot_general(td_ref[...], oh_d, (((0,), (0,)), ((), ())),
                              preferred_element_type=jnp.float32)
    acc = acc + jax.lax.dot_general(tw_ref[...], oh_w, (((0,), (0,)), ((), ())),
                                    preferred_element_type=jnp.float32)
    out_ref[0, :, :] = acc


def kernel(x, time_day, time_week):
    B, S, N, _ = x.shape
    T, F = time_day.shape
    xl = x[:, -1]
    tw_pad = jnp.zeros((8, F), jnp.float32).at[:7].set(time_week)

    out = pl.pallas_call(
        _body,
        grid=(B,),
        in_specs=[
            pl.BlockSpec((1, N, 3), lambda b: (b, 0, 0)),
            pl.BlockSpec((T, F), lambda b: (0, 0)),
            pl.BlockSpec((8, F), lambda b: (0, 0)),
        ],
        out_specs=pl.BlockSpec((1, F, N), lambda b: (b, 0, 0)),
        out_shape=jax.ShapeDtypeStruct((B, F, N), jnp.float32),
    )(xl, time_day, tw_pad)
    return out[..., None]
